# baseline (speedup 1.0000x reference)
import functools

import jax
import jax.numpy as jnp
from jax import lax
from jax.experimental import pallas as pl
from jax.experimental.pallas import tpu as pltpu
from jax.experimental.pallas import tpu_sc as plsc

N = 10000
E = 320000
D = 128
NC = 2
NS = 16
NW = NC * NS
C = 160
C2 = 80
NPAD = 10240
RPT = NPAD // NS
EPW = 10240
EPW_CHUNKS = EPW // C
EPAD = NW * EPW


def _sc_agg_body(with_deg, *refs):
    nh = C // C2
    if with_deg:
        (x_hbm, src_hbm, dst_hbm, agg_out, deg_out, *rest) = refs
        (srcs, d0, d1, d2, d3, rows0, rows1, ones_v, agg_sh, deg_sh,
         isem0, isem1, gsem0, gsem1, ssem0, ssem1) = (
            rest[:2], rest[2:2 + nh], rest[2 + nh:2 + 2 * nh],
            rest[2 + 2 * nh:2 + 3 * nh], rest[2 + 3 * nh:2 + 4 * nh],
            *rest[2 + 4 * nh:])
    else:
        (x_hbm, src_hbm, dst_hbm, agg_out, *rest) = refs
        (srcs, d0, d1, d2, d3, rows0, rows1, agg_sh,
         isem0, isem1, gsem0, gsem1, ssem0, ssem1) = (
            rest[:2], rest[2:2 + nh], rest[2 + nh:2 + 2 * nh],
            rest[2 + 2 * nh:2 + 3 * nh], rest[2 + 3 * nh:2 + 4 * nh],
            *rest[2 + 4 * nh:])
    dsts = (d0, d1, d2, d3)
    rows = (rows0, rows1)
    isem = (isem0, isem1)
    gsem = (gsem0, gsem1)
    ssem = (ssem0, ssem1)

    c = lax.axis_index("c")
    s = lax.axis_index("s")
    wid = s * NC + c
    ebase = pl.multiple_of(wid * EPW, 8)

    zero16 = jnp.zeros((16,), jnp.float32)

    def memset_row(i, carry):
        for k in range(D // 16):
            rows0[i, pl.ds(k * 16, 16)] = zero16
        return carry

    lax.fori_loop(0, 128, memset_row, 0)
    zrows = rows0 if C == 128 else rows0.at[pl.ds(0, 128)]
    for b in range(RPT // 128):
        pltpu.sync_copy(zrows, agg_sh.at[pl.ds(s * RPT + b * 128, 128)])
    if with_deg:
        one16 = jnp.ones((16,), jnp.float32)
        for k in range(C2 // 16):
            ones_v[pl.ds(k * 16, 16)] = one16
        for b in range(RPT // 128):
            pltpu.sync_copy(rows0.at[0], deg_sh.at[pl.ds(s * RPT + b * 128, 128)])

    def start_idx(j, q, b):
        base = pl.multiple_of(ebase + j * C, 8)
        pltpu.async_copy(src_hbm.at[pl.ds(base, C)], srcs[b], isem[b])
        for h in range(nh):
            pltpu.async_copy(
                dst_hbm.at[pl.ds(base + h * C2, C2)], dsts[q][h], isem[b])

    def wait_idx(b):
        pltpu.make_async_copy(
            src_hbm.at[pl.ds(0, C)], srcs[b], isem[b]).wait()
        for h in range(nh):
            pltpu.make_async_copy(
                dst_hbm.at[pl.ds(0, C2)], dsts[0][h], isem[b]).wait()

    def start_gather(b):
        pltpu.async_copy(x_hbm.at[srcs[b]], rows[b], gsem[b])

    def wait_gather(b):
        pltpu.make_async_copy(x_hbm.at[srcs[b]], rows[b], gsem[b]).wait()

    def rsrc(b, h):
        return rows[b] if nh == 1 else rows[b].at[pl.ds(h * C2, C2)]

    def start_scatter(q, b):
        for h in range(nh):
            pltpu.async_copy(rsrc(b, h), agg_sh.at[dsts[q][h]], ssem[b],
                             add=True)
        if with_deg:
            for h in range(nh):
                pltpu.async_copy(ones_v, deg_sh.at[dsts[q][h]], ssem[b],
                                 add=True)

    def wait_scatter(b):
        for h in range(nh):
            pltpu.make_async_copy(
                rsrc(b, h), agg_sh.at[dsts[0][h]], ssem[b]).wait()
        if with_deg:
            for h in range(nh):
                pltpu.make_async_copy(
                    ones_v, deg_sh.at[dsts[0][h]], ssem[b]).wait()

    start_idx(0, 0, 0)
    plsc.subcore_barrier()
    wait_idx(0)
    start_gather(0)
    start_idx(1, 1, 1)

    def quad_body(first, jj):
        for b4 in range(4):
            j = jj + b4
            b = b4 % 2
            wait_gather(b)
            wait_idx(b ^ 1)
            if not (first and b4 == 0):
                wait_scatter(b ^ 1)
            start_gather(b ^ 1)
            start_scatter(b4, b)
            start_idx(j + 2, (b4 + 2) % 4, b)

    quad_body(True, 0)

    def quad_steady(i, carry):
        quad_body(False, (i + 1) * 4)
        return carry

    lax.fori_loop(0, EPW_CHUNKS // 4 - 1, quad_steady, 0)
    wait_gather(EPW_CHUNKS % 2)
    wait_idx((EPW_CHUNKS + 1) % 2)
    wait_scatter((EPW_CHUNKS - 1) % 2)
    plsc.subcore_barrier()

    off = pl.multiple_of(s * RPT, 8)
    pltpu.sync_copy(agg_sh.at[pl.ds(off, RPT)], agg_out.at[c, pl.ds(off, RPT)])
    if with_deg:
        pltpu.sync_copy(deg_sh.at[pl.ds(off, RPT)], deg_out.at[c, pl.ds(off, RPT)])


def _make_sc_agg(with_deg):
    mesh = plsc.VectorSubcoreMesh(
        core_axis_name="c", subcore_axis_name="s",
        num_cores=NC, num_subcores=NS)
    if with_deg:
        out_type = (jax.ShapeDtypeStruct((NC, NPAD, D), jnp.float32),
                    jax.ShapeDtypeStruct((NC, NPAD), jnp.float32))
    else:
        out_type = jax.ShapeDtypeStruct((NC, NPAD, D), jnp.float32)
    scratch = [pltpu.VMEM((C,), jnp.int32)
               for _ in range(2)]
    scratch += [pltpu.VMEM((C2,), jnp.int32)
                for _ in range(4 * (C // C2))]
    scratch += [pltpu.VMEM((C, D), jnp.float32)
                for _ in range(2)]
    if with_deg:
        scratch.append(pltpu.VMEM((C2,), jnp.float32))
    scratch.append(pltpu.VMEM_SHARED((NPAD, D), jnp.float32))
    if with_deg:
        scratch.append(pltpu.VMEM_SHARED((NPAD,), jnp.float32))
    scratch.extend([pltpu.SemaphoreType.DMA] * 6)
    return pl.kernel(
        functools.partial(_sc_agg_body, with_deg),
        out_type=out_type, mesh=mesh, scratch_types=scratch)


_sc_agg_deg = _make_sc_agg(True)
_sc_agg = _make_sc_agg(False)

_R = 2000


def _dense_body(relu, x_ref, p0_ref, p1_ref, d0_ref, d1_ref,
                ws_ref, wn_ref, b_ref, o_ref):
    deg = d0_ref[...] + d1_ref[...]
    scale = 1.0 / jnp.maximum(deg, 1.0)
    agg = (p0_ref[...] + p1_ref[...]) * scale
    h = (jnp.dot(x_ref[...], ws_ref[...],
                 preferred_element_type=jnp.float32,
                 precision=lax.Precision.HIGHEST)
         + jnp.dot(agg, wn_ref[...],
                   preferred_element_type=jnp.float32,
                   precision=lax.Precision.HIGHEST)
         + b_ref[...])
    if relu:
        h = jnp.maximum(h, 0.0)
    o_ref[...] = h


def _dense(xa, aggp, degp, Ws, Wn, b, relu):
    d0 = degp[0].reshape(NPAD, 1)
    d1 = degp[1].reshape(NPAD, 1)
    return pl.pallas_call(
        functools.partial(_dense_body, relu),
        grid=(N // _R,),
        in_specs=[
            pl.BlockSpec((_R, D), lambda i: (i, 0)),
            pl.BlockSpec((_R, D), lambda i: (i, 0)),
            pl.BlockSpec((_R, D), lambda i: (i, 0)),
            pl.BlockSpec((_R, 1), lambda i: (i, 0)),
            pl.BlockSpec((_R, 1), lambda i: (i, 0)),
            pl.BlockSpec((D, D), lambda i: (0, 0)),
            pl.BlockSpec((D, D), lambda i: (0, 0)),
            pl.BlockSpec((1, D), lambda i: (0, 0)),
        ],
        out_specs=pl.BlockSpec((_R, D), lambda i: (i, 0)),
        out_shape=jax.ShapeDtypeStruct((N, D), jnp.float32),
    )(xa, aggp[0], aggp[1], d0, d1, Ws, Wn, b.reshape(1, D))


def kernel(x, W1s, W1n, b1, W2s, W2n, b2, W3s, W3n, b3, edge_index):
    src = edge_index[0].astype(jnp.int32)
    dst = edge_index[1].astype(jnp.int32)
    pad = EPAD + 2 * C - E
    src_p = jnp.concatenate([src, jnp.arange(pad, dtype=jnp.int32) % N])
    dst_p = jnp.concatenate(
        [dst, N + (jnp.arange(pad, dtype=jnp.int32) % (NPAD - N))])

    agg1, degp = _sc_agg_deg(x, src_p, dst_p)
    h1 = _dense(x, agg1, degp, W1s, W1n, b1, True)
    agg2 = _sc_agg(h1, src_p, dst_p)
    h2 = _dense(h1, agg2, degp, W2s, W2n, b2, True)
    agg3 = _sc_agg(h2, src_p, dst_p)
    return _dense(h2, agg3, degp, W3s, W3n, b3, False)


# speedup vs baseline: 1.0082x; 1.0082x over previous
import functools

import jax
import jax.numpy as jnp
from jax import lax
from jax.experimental import pallas as pl
from jax.experimental.pallas import tpu as pltpu
from jax.experimental.pallas import tpu_sc as plsc

N = 10000
E = 320000
D = 128
NC = 2
NS = 16
NW = NC * NS
C = 160
C2 = 160
NPAD = 10240
RPT = NPAD // NS
EPW = 10240
EPW_CHUNKS = EPW // C
EPAD = NW * EPW


def _sc_agg_body(with_deg, *refs):
    nh = C // C2
    if with_deg:
        (x_hbm, src_hbm, dst_hbm, agg_out, deg_out, *rest) = refs
        (srcs, d0, d1, d2, d3, rows0, rows1, ones_v, agg_sh, deg_sh,
         isem0, isem1, gsem0, gsem1, ssem0, ssem1) = (
            rest[:2], rest[2:2 + nh], rest[2 + nh:2 + 2 * nh],
            rest[2 + 2 * nh:2 + 3 * nh], rest[2 + 3 * nh:2 + 4 * nh],
            *rest[2 + 4 * nh:])
    else:
        (x_hbm, src_hbm, dst_hbm, agg_out, *rest) = refs
        (srcs, d0, d1, d2, d3, rows0, rows1, agg_sh,
         isem0, isem1, gsem0, gsem1, ssem0, ssem1) = (
            rest[:2], rest[2:2 + nh], rest[2 + nh:2 + 2 * nh],
            rest[2 + 2 * nh:2 + 3 * nh], rest[2 + 3 * nh:2 + 4 * nh],
            *rest[2 + 4 * nh:])
    dsts = (d0, d1, d2, d3)
    rows = (rows0, rows1)
    isem = (isem0, isem1)
    gsem = (gsem0, gsem1)
    ssem = (ssem0, ssem1)

    c = lax.axis_index("c")
    s = lax.axis_index("s")
    wid = s * NC + c
    ebase = pl.multiple_of(wid * EPW, 8)

    zero16 = jnp.zeros((16,), jnp.float32)

    def memset_row(i, carry):
        for k in range(D // 16):
            rows0[i, pl.ds(k * 16, 16)] = zero16
        return carry

    lax.fori_loop(0, 128, memset_row, 0)
    zrows = rows0 if C == 128 else rows0.at[pl.ds(0, 128)]
    for b in range(RPT // 128):
        pltpu.sync_copy(zrows, agg_sh.at[pl.ds(s * RPT + b * 128, 128)])
    if with_deg:
        one16 = jnp.ones((16,), jnp.float32)
        for k in range(C2 // 16):
            ones_v[pl.ds(k * 16, 16)] = one16
        for b in range(RPT // 128):
            pltpu.sync_copy(rows0.at[0], deg_sh.at[pl.ds(s * RPT + b * 128, 128)])

    def start_idx(j, q, b):
        base = pl.multiple_of(ebase + j * C, 8)
        pltpu.async_copy(src_hbm.at[pl.ds(base, C)], srcs[b], isem[b])
        for h in range(nh):
            pltpu.async_copy(
                dst_hbm.at[pl.ds(base + h * C2, C2)], dsts[q][h], isem[b])

    def wait_idx(b):
        pltpu.make_async_copy(
            src_hbm.at[pl.ds(0, C)], srcs[b], isem[b]).wait()
        for h in range(nh):
            pltpu.make_async_copy(
                dst_hbm.at[pl.ds(0, C2)], dsts[0][h], isem[b]).wait()

    def start_gather(b):
        pltpu.async_copy(x_hbm.at[srcs[b]], rows[b], gsem[b])

    def wait_gather(b):
        pltpu.make_async_copy(x_hbm.at[srcs[b]], rows[b], gsem[b]).wait()

    def rsrc(b, h):
        return rows[b] if nh == 1 else rows[b].at[pl.ds(h * C2, C2)]

    def start_scatter(q, b):
        for h in range(nh):
            pltpu.async_copy(rsrc(b, h), agg_sh.at[dsts[q][h]], ssem[b],
                             add=True)
        if with_deg:
            for h in range(nh):
                pltpu.async_copy(ones_v, deg_sh.at[dsts[q][h]], ssem[b],
                                 add=True)

    def wait_scatter(b):
        for h in range(nh):
            pltpu.make_async_copy(
                rsrc(b, h), agg_sh.at[dsts[0][h]], ssem[b]).wait()
        if with_deg:
            for h in range(nh):
                pltpu.make_async_copy(
                    ones_v, deg_sh.at[dsts[0][h]], ssem[b]).wait()

    start_idx(0, 0, 0)
    plsc.subcore_barrier()
    wait_idx(0)
    start_gather(0)
    start_idx(1, 1, 1)

    def quad_body(first, jj):
        for b4 in range(4):
            j = jj + b4
            b = b4 % 2
            wait_gather(b)
            wait_idx(b ^ 1)
            if not (first and b4 == 0):
                wait_scatter(b ^ 1)
            start_gather(b ^ 1)
            start_scatter(b4, b)
            start_idx(j + 2, (b4 + 2) % 4, b)

    quad_body(True, 0)

    def quad_steady(i, carry):
        quad_body(False, (i + 1) * 4)
        return carry

    lax.fori_loop(0, EPW_CHUNKS // 4 - 1, quad_steady, 0)
    wait_gather(EPW_CHUNKS % 2)
    wait_idx((EPW_CHUNKS + 1) % 2)
    wait_scatter((EPW_CHUNKS - 1) % 2)
    plsc.subcore_barrier()

    off = pl.multiple_of(s * RPT, 8)
    pltpu.sync_copy(agg_sh.at[pl.ds(off, RPT)], agg_out.at[c, pl.ds(off, RPT)])
    if with_deg:
        pltpu.sync_copy(deg_sh.at[pl.ds(off, RPT)], deg_out.at[c, pl.ds(off, RPT)])


def _make_sc_agg(with_deg):
    mesh = plsc.VectorSubcoreMesh(
        core_axis_name="c", subcore_axis_name="s",
        num_cores=NC, num_subcores=NS)
    if with_deg:
        out_type = (jax.ShapeDtypeStruct((NC, NPAD, D), jnp.float32),
                    jax.ShapeDtypeStruct((NC, NPAD), jnp.float32))
    else:
        out_type = jax.ShapeDtypeStruct((NC, NPAD, D), jnp.float32)
    scratch = [pltpu.VMEM((C,), jnp.int32)
               for _ in range(2)]
    scratch += [pltpu.VMEM((C2,), jnp.int32)
                for _ in range(4 * (C // C2))]
    scratch += [pltpu.VMEM((C, D), jnp.float32)
                for _ in range(2)]
    if with_deg:
        scratch.append(pltpu.VMEM((C2,), jnp.float32))
    scratch.append(pltpu.VMEM_SHARED((NPAD, D), jnp.float32))
    if with_deg:
        scratch.append(pltpu.VMEM_SHARED((NPAD,), jnp.float32))
    scratch.extend([pltpu.SemaphoreType.DMA] * 6)
    return pl.kernel(
        functools.partial(_sc_agg_body, with_deg),
        out_type=out_type, mesh=mesh, scratch_types=scratch)


_sc_agg_deg = _make_sc_agg(True)
_sc_agg = _make_sc_agg(False)

_R = 2000


def _dense_body(relu, x_ref, p0_ref, p1_ref, d0_ref, d1_ref,
                ws_ref, wn_ref, b_ref, o_ref):
    deg = d0_ref[...] + d1_ref[...]
    scale = 1.0 / jnp.maximum(deg, 1.0)
    agg = (p0_ref[...] + p1_ref[...]) * scale
    h = (jnp.dot(x_ref[...], ws_ref[...],
                 preferred_element_type=jnp.float32,
                 precision=lax.Precision.HIGHEST)
         + jnp.dot(agg, wn_ref[...],
                   preferred_element_type=jnp.float32,
                   precision=lax.Precision.HIGHEST)
         + b_ref[...])
    if relu:
        h = jnp.maximum(h, 0.0)
    o_ref[...] = h


def _dense(xa, aggp, degp, Ws, Wn, b, relu):
    d0 = degp[0].reshape(NPAD, 1)
    d1 = degp[1].reshape(NPAD, 1)
    return pl.pallas_call(
        functools.partial(_dense_body, relu),
        grid=(N // _R,),
        in_specs=[
            pl.BlockSpec((_R, D), lambda i: (i, 0)),
            pl.BlockSpec((_R, D), lambda i: (i, 0)),
            pl.BlockSpec((_R, D), lambda i: (i, 0)),
            pl.BlockSpec((_R, 1), lambda i: (i, 0)),
            pl.BlockSpec((_R, 1), lambda i: (i, 0)),
            pl.BlockSpec((D, D), lambda i: (0, 0)),
            pl.BlockSpec((D, D), lambda i: (0, 0)),
            pl.BlockSpec((1, D), lambda i: (0, 0)),
        ],
        out_specs=pl.BlockSpec((_R, D), lambda i: (i, 0)),
        out_shape=jax.ShapeDtypeStruct((N, D), jnp.float32),
    )(xa, aggp[0], aggp[1], d0, d1, Ws, Wn, b.reshape(1, D))


def kernel(x, W1s, W1n, b1, W2s, W2n, b2, W3s, W3n, b3, edge_index):
    src = edge_index[0].astype(jnp.int32)
    dst = edge_index[1].astype(jnp.int32)
    pad = EPAD + 2 * C - E
    src_p = jnp.concatenate([src, jnp.arange(pad, dtype=jnp.int32) % N])
    dst_p = jnp.concatenate(
        [dst, N + (jnp.arange(pad, dtype=jnp.int32) % (NPAD - N))])

    agg1, degp = _sc_agg_deg(x, src_p, dst_p)
    h1 = _dense(x, agg1, degp, W1s, W1n, b1, True)
    agg2 = _sc_agg(h1, src_p, dst_p)
    h2 = _dense(h1, agg2, degp, W2s, W2n, b2, True)
    agg3 = _sc_agg(h2, src_p, dst_p)
    return _dense(h2, agg3, degp, W3s, W3n, b3, False)
